# baseline (speedup 1.0000x reference)
import functools

import jax
import jax.numpy as jnp
from jax.experimental import pallas as pl
from jax.experimental.pallas import tpu as pltpu

NUM_GENES = 18211
INTER_DIM = 1024
LATENT_DIM = 128
BATCH = 64

KBLK = 2560
NK = (NUM_GENES + KBLK - 1) // KBLK
HALF = INTER_DIM // 2


def _mlp_kernel(x_ref, w1a_ref, w1b_ref, b1_ref, w2_ref, b2_ref, z_ref,
                acc_ref):
    s = pl.program_id(0)
    x_blk = x_ref[...]

    @pl.when(s == 0)
    def _first():
        base = (NK - 1) * KBLK
        col_ids = jax.lax.broadcasted_iota(jnp.int32, (BATCH, KBLK), 1)
        xm = jnp.where(base + col_ids < NUM_GENES, x_blk, 0.0)
        row_ids = jax.lax.broadcasted_iota(jnp.int32, (KBLK, 1), 0)
        rmask = base + row_ids < NUM_GENES
        acc_ref[:, :HALF] = jnp.dot(
            xm, jnp.where(rmask, w1a_ref[...], 0.0),
            preferred_element_type=jnp.float32,
            precision=jax.lax.Precision.DEFAULT,
        )
        acc_ref[:, HALF:] = jnp.dot(
            xm, jnp.where(rmask, w1b_ref[...], 0.0),
            preferred_element_type=jnp.float32,
            precision=jax.lax.Precision.DEFAULT,
        )

    @pl.when(s > 0)
    def _accum():
        acc_ref[:, :HALF] += jnp.dot(
            x_blk, w1a_ref[...],
            preferred_element_type=jnp.float32,
            precision=jax.lax.Precision.DEFAULT,
        )
        acc_ref[:, HALF:] += jnp.dot(
            x_blk, w1b_ref[...],
            preferred_element_type=jnp.float32,
            precision=jax.lax.Precision.DEFAULT,
        )

    @pl.when(s == NK - 1)
    def _finish():
        h = acc_ref[...] + b1_ref[...]
        h = jnp.where(h > 0, h, 0.25 * h)
        z = jnp.dot(h, w2_ref[...], preferred_element_type=jnp.float32)
        z_ref[...] = z + b2_ref[...]


def _kidx(s):
    return jax.lax.rem(s + NK - 1, NK)


@functools.partial(jax.jit, static_argnames=())
def kernel(x, W1, b1, W2, b2):
    b1r = b1.reshape(1, INTER_DIM)
    b2r = b2.reshape(1, LATENT_DIM)
    return pl.pallas_call(
        _mlp_kernel,
        grid=(NK,),
        in_specs=[
            pl.BlockSpec((BATCH, KBLK), lambda s: (0, _kidx(s))),
            pl.BlockSpec((KBLK, HALF), lambda s: (_kidx(s), 0)),
            pl.BlockSpec((KBLK, HALF), lambda s: (_kidx(s), 1)),
            pl.BlockSpec((1, INTER_DIM), lambda s: (0, 0)),
            pl.BlockSpec((INTER_DIM, LATENT_DIM), lambda s: (0, 0)),
            pl.BlockSpec((1, LATENT_DIM), lambda s: (0, 0)),
        ],
        out_specs=pl.BlockSpec((BATCH, LATENT_DIM), lambda s: (0, 0)),
        out_shape=jax.ShapeDtypeStruct((BATCH, LATENT_DIM), jnp.float32),
        scratch_shapes=[pltpu.VMEM((BATCH, INTER_DIM), jnp.float32)],
    )(x, W1, W1, b1r, W2, b2r)


# speedup vs baseline: 1.0273x; 1.0273x over previous
import functools

import jax
import jax.numpy as jnp
from jax.experimental import pallas as pl
from jax.experimental.pallas import tpu as pltpu

NUM_GENES = 18211
INTER_DIM = 1024
LATENT_DIM = 128
BATCH = 64

KBLK = 2048
NK = (NUM_GENES + KBLK - 1) // KBLK
HALF = INTER_DIM // 2


def _mlp_kernel(x_ref, w1a_ref, w1b_ref, b1_ref, w2_ref, b2_ref, z_ref,
                acc_ref):
    s = pl.program_id(0)
    x_blk = x_ref[...]

    @pl.when(s == 0)
    def _first():
        base = (NK - 1) * KBLK
        col_ids = jax.lax.broadcasted_iota(jnp.int32, (BATCH, KBLK), 1)
        xm = jnp.where(base + col_ids < NUM_GENES, x_blk, 0.0)
        row_ids = jax.lax.broadcasted_iota(jnp.int32, (KBLK, 1), 0)
        rmask = base + row_ids < NUM_GENES
        acc_ref[:, :HALF] = jnp.dot(
            xm, jnp.where(rmask, w1a_ref[...], 0.0),
            preferred_element_type=jnp.float32,
            precision=jax.lax.Precision.DEFAULT,
        )
        acc_ref[:, HALF:] = jnp.dot(
            xm, jnp.where(rmask, w1b_ref[...], 0.0),
            preferred_element_type=jnp.float32,
            precision=jax.lax.Precision.DEFAULT,
        )

    @pl.when(s > 0)
    def _accum():
        acc_ref[:, :HALF] += jnp.dot(
            x_blk, w1a_ref[...],
            preferred_element_type=jnp.float32,
            precision=jax.lax.Precision.DEFAULT,
        )
        acc_ref[:, HALF:] += jnp.dot(
            x_blk, w1b_ref[...],
            preferred_element_type=jnp.float32,
            precision=jax.lax.Precision.DEFAULT,
        )

    @pl.when(s == NK - 1)
    def _finish():
        h = acc_ref[...] + b1_ref[...]
        h = jnp.where(h > 0, h, 0.25 * h)
        z = jnp.dot(h, w2_ref[...], preferred_element_type=jnp.float32)
        z_ref[...] = z + b2_ref[...]


def _kidx(s):
    return jax.lax.rem(s + NK - 1, NK)


@functools.partial(jax.jit, static_argnames=())
def kernel(x, W1, b1, W2, b2):
    b1r = b1.reshape(1, INTER_DIM)
    b2r = b2.reshape(1, LATENT_DIM)
    return pl.pallas_call(
        _mlp_kernel,
        grid=(NK,),
        in_specs=[
            pl.BlockSpec((BATCH, KBLK), lambda s: (0, _kidx(s))),
            pl.BlockSpec((KBLK, HALF), lambda s: (_kidx(s), 0)),
            pl.BlockSpec((KBLK, HALF), lambda s: (_kidx(s), 1)),
            pl.BlockSpec((1, INTER_DIM), lambda s: (0, 0)),
            pl.BlockSpec((INTER_DIM, LATENT_DIM), lambda s: (0, 0)),
            pl.BlockSpec((1, LATENT_DIM), lambda s: (0, 0)),
        ],
        out_specs=pl.BlockSpec((BATCH, LATENT_DIM), lambda s: (0, 0)),
        out_shape=jax.ShapeDtypeStruct((BATCH, LATENT_DIM), jnp.float32),
        scratch_shapes=[pltpu.VMEM((BATCH, INTER_DIM), jnp.float32)],
    )(x, W1, W1, b1r, W2, b2r)
